# Initial kernel scaffold; baseline (speedup 1.0000x reference)
import functools

import jax
import jax.numpy as jnp
from jax import lax
from jax.experimental import pallas as pl
from jax.experimental.pallas import tpu as pltpu
from jax.experimental.pallas import tpu_sc as plsc

_EMBED = 32


@functools.lru_cache(maxsize=None)
def _make_kernel(n_rows: int, n_tags: int, d: int):
    info = plsc.get_sparse_core_info()
    nc, ns = info.num_cores, info.num_subcores
    nw = nc * ns
    per_w = n_rows // nw
    chunk = 1280
    n_chunks = per_w // chunk
    sub = 128
    n_sub = chunk // sub
    assert per_w % chunk == 0 and chunk % sub == 0

    mesh = plsc.VectorSubcoreMesh(core_axis_name="c", subcore_axis_name="s")

    @functools.partial(
        pl.kernel,
        mesh=mesh,
        out_type=jax.ShapeDtypeStruct((n_rows, d), jnp.float32),
        scratch_types=[
            pltpu.VMEM((chunk,), jnp.int32),
            pltpu.VMEM((chunk, d), jnp.float32),
            pltpu.SemaphoreType.DMA,
        ],
    )
    def k(x_hbm, tab_hbm, out_hbm, idx_v, rows_v, sem):
        wid = lax.axis_index("s") * nc + lax.axis_index("c")
        base_w = wid * per_w

        def body(g, carry):
            base = base_w + g * chunk
            pltpu.sync_copy(x_hbm.at[pl.ds(base, chunk)], idx_v)
            for j in range(n_sub):
                pltpu.async_copy(
                    tab_hbm.at[idx_v.at[pl.ds(j * sub, sub)]],
                    rows_v.at[pl.ds(j * sub, sub)],
                    sem,
                )
            for j in range(n_sub):
                pltpu.make_async_copy(
                    tab_hbm.at[idx_v.at[pl.ds(j * sub, sub)]],
                    rows_v.at[pl.ds(j * sub, sub)],
                    sem,
                ).wait()
            pltpu.sync_copy(rows_v, out_hbm.at[pl.ds(base, chunk)])
            return carry

        lax.fori_loop(0, n_chunks, body, 0)

    return k


def kernel(x, embed_weight):
    b, s = x.shape
    n_rows = b * s
    flat = x.reshape(n_rows).astype(jnp.int32)
    tab = embed_weight.astype(jnp.float32)
    out = _make_kernel(n_rows, tab.shape[0], tab.shape[1])(flat, tab)
    return out.reshape(b, s, _EMBED)


# speedup vs baseline: 5.1062x; 5.1062x over previous
import functools

import jax
import jax.numpy as jnp
from jax import lax
from jax.experimental import pallas as pl
from jax.experimental.pallas import tpu as pltpu
from jax.experimental.pallas import tpu_sc as plsc

_EMBED = 32


@functools.lru_cache(maxsize=None)
def _make_kernel(n_rows: int, n_tags: int, d: int):
    info = plsc.get_sparse_core_info()
    nc, ns = info.num_cores, info.num_subcores
    nw = nc * ns
    per_w = n_rows // nw
    chunk = 1280
    n_chunks = per_w // chunk
    sub = 128
    n_sub = chunk // sub
    assert per_w % chunk == 0 and chunk % sub == 0

    mesh = plsc.VectorSubcoreMesh(core_axis_name="c", subcore_axis_name="s")

    @functools.partial(
        pl.kernel,
        mesh=mesh,
        out_type=jax.ShapeDtypeStruct((n_rows, d), jnp.float32),
        scratch_types=[
            pltpu.VMEM((chunk,), jnp.int32),
            pltpu.VMEM((chunk, d), jnp.float32),
            pltpu.SemaphoreType.DMA,
        ],
        compiler_params=pltpu.CompilerParams(use_tc_tiling_on_sc=False),
    )
    def k(x_hbm, tab_hbm, out_hbm, idx_v, rows_v, sem):
        wid = lax.axis_index("s") * nc + lax.axis_index("c")
        base_w = wid * per_w

        def body(g, carry):
            base = base_w + g * chunk
            pltpu.sync_copy(x_hbm.at[pl.ds(base, chunk)], idx_v)
            for j in range(n_sub):
                pltpu.async_copy(
                    tab_hbm.at[idx_v.at[pl.ds(j * sub, sub)]],
                    rows_v.at[pl.ds(j * sub, sub)],
                    sem,
                )
            for j in range(n_sub):
                pltpu.make_async_copy(
                    tab_hbm.at[idx_v.at[pl.ds(j * sub, sub)]],
                    rows_v.at[pl.ds(j * sub, sub)],
                    sem,
                ).wait()
            pltpu.sync_copy(rows_v, out_hbm.at[pl.ds(base, chunk)])
            return carry

        lax.fori_loop(0, n_chunks, body, 0)

    return k


def kernel(x, embed_weight):
    b, s = x.shape
    n_rows = b * s
    flat = x.reshape(n_rows).astype(jnp.int32)
    tab = embed_weight.astype(jnp.float32)
    out = _make_kernel(n_rows, tab.shape[0], tab.shape[1])(flat, tab)
    return out.reshape(b, s, _EMBED)


# speedup vs baseline: 5.2719x; 1.0324x over previous
import functools

import jax
import jax.numpy as jnp
from jax import lax
from jax.experimental import pallas as pl
from jax.experimental.pallas import tpu as pltpu
from jax.experimental.pallas import tpu_sc as plsc

_EMBED = 32


@functools.lru_cache(maxsize=None)
def _make_kernel(n_rows: int, n_tags: int, d: int):
    info = plsc.get_sparse_core_info()
    nc, ns = info.num_cores, info.num_subcores
    nw = nc * ns
    per_w = n_rows // nw
    chunk = 1280
    n_chunks = per_w // chunk
    sub = 128
    n_sub = chunk // sub
    assert per_w % chunk == 0 and chunk % sub == 0 and n_chunks % 2 == 0

    mesh = plsc.VectorSubcoreMesh(core_axis_name="c", subcore_axis_name="s")

    @functools.partial(
        pl.kernel,
        mesh=mesh,
        out_type=jax.ShapeDtypeStruct((n_rows, d), jnp.float32),
        scratch_types=[
            pltpu.VMEM((chunk,), jnp.int32),
            pltpu.VMEM((chunk,), jnp.int32),
            pltpu.VMEM((chunk, d), jnp.float32),
            pltpu.VMEM((chunk, d), jnp.float32),
            pltpu.SemaphoreType.DMA,
            pltpu.SemaphoreType.DMA,
            pltpu.SemaphoreType.DMA,
            pltpu.SemaphoreType.DMA,
            pltpu.SemaphoreType.DMA,
            pltpu.SemaphoreType.DMA,
        ],
        compiler_params=pltpu.CompilerParams(use_tc_tiling_on_sc=False),
    )
    def k(x_hbm, tab_hbm, out_hbm, idx0, idx1, rows0, rows1,
          si0, si1, sg0, sg1, so0, so1):
        idx_v = (idx0, idx1)
        rows_v = (rows0, rows1)
        sem_idx = (si0, si1)
        sem_gat = (sg0, sg1)
        sem_out = (so0, so1)

        wid = lax.axis_index("s") * nc + lax.axis_index("c")
        base_w = wid * per_w

        def idx_copy(c, b):
            return pltpu.make_async_copy(
                x_hbm.at[pl.ds(base_w + c * chunk, chunk)], idx_v[b],
                sem_idx[b])

        def gather_copy(b, j):
            return pltpu.make_async_copy(
                tab_hbm.at[idx_v[b].at[pl.ds(j * sub, sub)]],
                rows_v[b].at[pl.ds(j * sub, sub)], sem_gat[b])

        def store_copy(c, b):
            return pltpu.make_async_copy(
                rows_v[b], out_hbm.at[pl.ds(base_w + c * chunk, chunk)],
                sem_out[b])

        def process(c, b, first):
            if not first:
                store_copy(c, b).wait()
            idx_copy(c, b).wait()
            for j in range(n_sub):
                gather_copy(b, j).start()
            for j in range(n_sub):
                gather_copy(b, j).wait()
            store_copy(c, b).start()
            @pl.when(c + 2 < n_chunks)
            def _():
                idx_copy(c + 2, b).start()

        idx_copy(0, 0).start()
        idx_copy(1, 1).start()
        process(0, 0, first=True)
        process(1, 1, first=True)

        def body(kk, carry):
            process(2 * kk, 0, first=False)
            process(2 * kk + 1, 1, first=False)
            return carry

        lax.fori_loop(1, n_chunks // 2, body, 0)
        store_copy(n_chunks - 2, 0).wait()
        store_copy(n_chunks - 1, 1).wait()

    return k


def kernel(x, embed_weight):
    b, s = x.shape
    n_rows = b * s
    flat = x.reshape(n_rows).astype(jnp.int32)
    tab = embed_weight.astype(jnp.float32)
    out = _make_kernel(n_rows, tab.shape[0], tab.shape[1])(flat, tab)
    return out.reshape(b, s, _EMBED)


# speedup vs baseline: 8.5273x; 1.6175x over previous
import functools

import jax
import jax.numpy as jnp
from jax import lax
from jax.experimental import pallas as pl
from jax.experimental.pallas import tpu as pltpu
from jax.experimental.pallas import tpu_sc as plsc


@functools.lru_cache(maxsize=None)
def _make_gather(n_rows: int, n_tags: int, d: int):
    info = plsc.get_sparse_core_info()
    nc, ns = info.num_cores, info.num_subcores
    nw = nc * ns
    per_w = n_rows // nw
    chunk = 1280
    n_chunks = per_w // chunk
    sub = 128
    n_sub = chunk // sub
    assert per_w % chunk == 0 and chunk % sub == 0 and n_chunks % 2 == 0

    mesh = plsc.VectorSubcoreMesh(core_axis_name="c", subcore_axis_name="s")

    @functools.partial(
        pl.kernel,
        mesh=mesh,
        out_type=jax.ShapeDtypeStruct((n_rows, d), jnp.float32),
        scratch_types=[
            pltpu.VMEM((chunk,), jnp.int32),
            pltpu.VMEM((chunk,), jnp.int32),
            pltpu.VMEM((chunk, d), jnp.float32),
            pltpu.VMEM((chunk, d), jnp.float32),
            pltpu.SemaphoreType.DMA,
            pltpu.SemaphoreType.DMA,
            pltpu.SemaphoreType.DMA,
            pltpu.SemaphoreType.DMA,
            pltpu.SemaphoreType.DMA,
            pltpu.SemaphoreType.DMA,
        ],
        compiler_params=pltpu.CompilerParams(use_tc_tiling_on_sc=False),
    )
    def k(x_hbm, tab_hbm, out_hbm, idx0, idx1, rows0, rows1,
          si0, si1, sg0, sg1, so0, so1):
        idx_v = (idx0, idx1)
        rows_v = (rows0, rows1)
        sem_idx = (si0, si1)
        sem_gat = (sg0, sg1)
        sem_out = (so0, so1)

        wid = lax.axis_index("s") * nc + lax.axis_index("c")
        base_w = wid * per_w

        def idx_copy(c, b):
            return pltpu.make_async_copy(
                x_hbm.at[pl.ds(base_w + c * chunk, chunk)], idx_v[b],
                sem_idx[b])

        def gather_copy(b, j):
            return pltpu.make_async_copy(
                tab_hbm.at[idx_v[b].at[pl.ds(j * sub, sub)]],
                rows_v[b].at[pl.ds(j * sub, sub)], sem_gat[b])

        def store_copy(c, b):
            return pltpu.make_async_copy(
                rows_v[b], out_hbm.at[pl.ds(base_w + c * chunk, chunk)],
                sem_out[b])

        def process(c, b, first):
            if not first:
                store_copy(c, b).wait()
            idx_copy(c, b).wait()
            for j in range(n_sub):
                gather_copy(b, j).start()
            for j in range(n_sub):
                gather_copy(b, j).wait()
            store_copy(c, b).start()

            @pl.when(c + 2 < n_chunks)
            def _():
                idx_copy(c + 2, b).start()

        idx_copy(0, 0).start()
        idx_copy(1, 1).start()
        process(0, 0, first=True)
        process(1, 1, first=True)

        def body(kk, carry):
            process(2 * kk, 0, first=False)
            process(2 * kk + 1, 1, first=False)
            return carry

        lax.fori_loop(1, n_chunks // 2, body, 0)
        store_copy(n_chunks - 2, 0).wait()
        store_copy(n_chunks - 1, 1).wait()

    return k


@functools.lru_cache(maxsize=None)
def _make_transpose(batch: int, seq: int, d: int):
    g = 128 // d
    sg = seq // g
    tb = batch // 128
    te = d // 8

    def body(in_ref, out_ref):
        for gi in range(sg):
            m = in_ref[0, pl.ds(gi * 128, 128), :]
            mt = jnp.transpose(m, (1, 0))
            out_ref[pl.ds(g * gi, g), :, 0, :, :] = mt.reshape(g, te, 8, 128)

    return pl.pallas_call(
        body,
        grid=(tb,),
        in_specs=[pl.BlockSpec((1, sg * 128, 128), lambda i: (i, 0, 0))],
        out_specs=pl.BlockSpec((seq, te, 1, 8, 128), lambda i: (0, 0, i, 0, 0)),
        out_shape=jax.ShapeDtypeStruct((seq, te, tb, 8, 128), jnp.float32),
    )


def kernel(x, embed_weight):
    b, s = x.shape
    n_tags, d = embed_weight.shape
    g = 128 // d
    sg = s // g
    tb = b // 128
    xp = (x.astype(jnp.int32)
           .reshape(tb, 128, sg, g)
           .transpose(0, 2, 1, 3)
           .reshape(b * s))
    tab = embed_weight.astype(jnp.float32)
    p1 = _make_gather(b * s, n_tags, d)(xp, tab)
    p3 = p1.reshape(tb, sg * 128, 128)
    o5 = _make_transpose(b, s, d)(p3)
    return o5.transpose(2, 4, 0, 1, 3).reshape(b, s, d)


# speedup vs baseline: 12.1140x; 1.4206x over previous
import functools

import jax
import jax.numpy as jnp
from jax import lax
from jax.experimental import pallas as pl
from jax.experimental.pallas import tpu as pltpu
from jax.experimental.pallas import tpu_sc as plsc


@functools.lru_cache(maxsize=None)
def _make_gather(n_rows: int, n_tags: int, d: int, seq: int):
    info = plsc.get_sparse_core_info()
    nc, ns = info.num_cores, info.num_subcores
    nw = nc * ns
    per_w = n_rows // nw
    g = 128 // d
    sg = seq // g
    chunk = 128 * g
    sub = 128
    n_sub = chunk // sub
    assert per_w == sg * chunk and sg % 2 == 0

    mesh = plsc.VectorSubcoreMesh(core_axis_name="c", subcore_axis_name="s")

    @functools.partial(
        pl.kernel,
        mesh=mesh,
        out_type=jax.ShapeDtypeStruct((n_rows, d), jnp.float32),
        scratch_types=[
            pltpu.VMEM((per_w,), jnp.int32),
            pltpu.VMEM((per_w,), jnp.int32),
            pltpu.VMEM((chunk, d), jnp.float32),
            pltpu.VMEM((chunk, d), jnp.float32),
            pltpu.SemaphoreType.DMA,
            pltpu.SemaphoreType.DMA,
            pltpu.SemaphoreType.DMA,
            pltpu.SemaphoreType.DMA,
        ],
        compiler_params=pltpu.CompilerParams(
            use_tc_tiling_on_sc=False, needs_layout_passes=False),
    )
    def k(x_hbm, tab_hbm, out_hbm, xblk, idx_all, rows0, rows1,
          sg0, sg1, so0, so1):
        rows_v = (rows0, rows1)
        sem_gat = (sg0, sg1)
        sem_out = (so0, so1)

        wid = lax.axis_index("s") * nc + lax.axis_index("c")
        base_w = wid * per_w

        pltpu.sync_copy(x_hbm.at[pl.ds(base_w, per_w)], xblk)

        iota = lax.iota(jnp.int32, 16)
        def build_all(v, carry):
            q = iota + v * 16
            gi = q >> 9
            r = q - gi * chunk
            off = (r >> 2) * seq + (r & (g - 1)) + gi * g
            vals = plsc.load_gather(xblk, [off])
            idx_all[pl.ds(v * 16, 16)] = vals
            return carry

        lax.fori_loop(0, per_w // 16, build_all, 0)

        def gather_copy(c, b, j):
            return pltpu.make_async_copy(
                tab_hbm.at[idx_all.at[pl.ds(c * chunk + j * sub, sub)]],
                rows_v[b].at[pl.ds(j * sub, sub)], sem_gat[b])

        def store_copy(c, b):
            return pltpu.make_async_copy(
                rows_v[b], out_hbm.at[pl.ds(base_w + c * chunk, chunk)],
                sem_out[b])

        def process(c, b, first):
            if not first:
                store_copy(c, b).wait()
            for j in range(n_sub):
                gather_copy(c, b, j).start()
            for j in range(n_sub):
                gather_copy(c, b, j).wait()
            store_copy(c, b).start()

        process(0, 0, first=True)
        process(1, 1, first=True)

        def body(kk, carry):
            process(2 * kk, 0, first=False)
            process(2 * kk + 1, 1, first=False)
            return carry

        lax.fori_loop(1, sg // 2, body, 0)
        store_copy(sg - 2, 0).wait()
        store_copy(sg - 1, 1).wait()

    return k


@functools.lru_cache(maxsize=None)
def _make_transpose(batch: int, seq: int, d: int):
    g = 128 // d
    sg = seq // g
    tb = batch // 128
    te = d // 8

    def body(in_ref, out_ref):
        for gi in range(sg):
            m = in_ref[0, pl.ds(gi * 128, 128), :]
            mt = jnp.transpose(m, (1, 0))
            out_ref[pl.ds(g * gi, g), :, 0, :, :] = mt.reshape(g, te, 8, 128)

    return pl.pallas_call(
        body,
        grid=(tb,),
        in_specs=[pl.BlockSpec((1, sg * 128, 128), lambda i: (i, 0, 0))],
        out_specs=pl.BlockSpec((seq, te, 1, 8, 128), lambda i: (0, 0, i, 0, 0)),
        out_shape=jax.ShapeDtypeStruct((seq, te, tb, 8, 128), jnp.float32),
    )


def kernel(x, embed_weight):
    b, s = x.shape
    n_tags, d = embed_weight.shape
    g = 128 // d
    sg = s // g
    tb = b // 128
    flat = x.reshape(b * s).astype(jnp.int32)
    tab = embed_weight.astype(jnp.float32)
    p1 = _make_gather(b * s, n_tags, d, s)(flat, tab)
    p3 = p1.reshape(tb, sg * 128, 128)
    o5 = _make_transpose(b, s, d)(p3)
    return o5.transpose(2, 4, 0, 1, 3).reshape(b, s, d)


# speedup vs baseline: 12.4961x; 1.0315x over previous
import functools

import jax
import jax.numpy as jnp
from jax import lax
from jax.experimental import pallas as pl
from jax.experimental.pallas import tpu as pltpu
from jax.experimental.pallas import tpu_sc as plsc


@functools.lru_cache(maxsize=None)
def _make_gather(n_rows: int, n_tags: int, d: int, seq: int):
    info = plsc.get_sparse_core_info()
    nc, ns = info.num_cores, info.num_subcores
    nw = nc * ns
    per_w = n_rows // nw
    g = 128 // d
    sg = seq // g
    chunk = 128 * g
    sub = 128
    n_sub = chunk // sub
    assert per_w == sg * chunk and sg % 2 == 0

    mesh = plsc.VectorSubcoreMesh(core_axis_name="c", subcore_axis_name="s")

    @functools.partial(
        pl.kernel,
        mesh=mesh,
        out_type=jax.ShapeDtypeStruct((n_rows, d), jnp.float32),
        scratch_types=[
            pltpu.VMEM((per_w,), jnp.int32),
            pltpu.VMEM((per_w,), jnp.int32),
            pltpu.VMEM((chunk, d), jnp.float32),
            pltpu.VMEM((chunk, d), jnp.float32),
            pltpu.SemaphoreType.DMA,
            pltpu.SemaphoreType.DMA,
            pltpu.SemaphoreType.DMA,
            pltpu.SemaphoreType.DMA,
        ],
        compiler_params=pltpu.CompilerParams(
            use_tc_tiling_on_sc=False, needs_layout_passes=False),
    )
    def k(x_hbm, tab_hbm, out_hbm, xblk, idx_all, rows0, rows1,
          sg0, sg1, so0, so1):
        rows_v = (rows0, rows1)
        sem_gat = (sg0, sg1)
        sem_out = (so0, so1)

        wid = lax.axis_index("s") * nc + lax.axis_index("c")
        base_w = wid * per_w

        pltpu.sync_copy(x_hbm.at[pl.ds(base_w, per_w)], xblk)

        iota = lax.iota(jnp.int32, 16)
        base_off = (iota >> 2) * seq + (iota & (g - 1))

        def build_group(c):
            def bg(v, carry):
                off = base_off + v * (4 * seq) + c * g
                vals = plsc.load_gather(xblk, [off])
                idx_all[pl.ds(c * chunk + v * 16, 16)] = vals
                return carry

            lax.fori_loop(0, chunk // 16, bg, 0)

        def gather_copy(c, b, j):
            return pltpu.make_async_copy(
                tab_hbm.at[idx_all.at[pl.ds(c * chunk + j * sub, sub)]],
                rows_v[b].at[pl.ds(j * sub, sub)], sem_gat[b])

        def store_copy(c, b):
            return pltpu.make_async_copy(
                rows_v[b], out_hbm.at[pl.ds(base_w + c * chunk, chunk)],
                sem_out[b])

        def process(c, b, first):
            if not first:
                store_copy(c, b).wait()
            for j in range(n_sub):
                gather_copy(c, b, j).start()
            @pl.when(c + 2 < sg)
            def _():
                build_group(c + 2)

            for j in range(n_sub):
                gather_copy(c, b, j).wait()
            store_copy(c, b).start()

        build_group(0)
        build_group(1)
        process(0, 0, first=True)
        process(1, 1, first=True)

        def body(kk, carry):
            process(2 * kk, 0, first=False)
            process(2 * kk + 1, 1, first=False)
            return carry

        lax.fori_loop(1, sg // 2, body, 0)
        store_copy(sg - 2, 0).wait()
        store_copy(sg - 1, 1).wait()

    return k


@functools.lru_cache(maxsize=None)
def _make_transpose(batch: int, seq: int, d: int):
    g = 128 // d
    sg = seq // g
    tb = batch // 128
    te = d // 8

    def body(in_ref, out_ref):
        for gi in range(sg):
            m = in_ref[0, pl.ds(gi * 128, 128), :]
            mt = jnp.transpose(m, (1, 0))
            out_ref[pl.ds(g * gi, g), :, 0, :, :] = mt.reshape(g, te, 8, 128)

    return pl.pallas_call(
        body,
        grid=(tb,),
        in_specs=[pl.BlockSpec((1, sg * 128, 128), lambda i: (i, 0, 0))],
        out_specs=pl.BlockSpec((seq, te, 1, 8, 128), lambda i: (0, 0, i, 0, 0)),
        out_shape=jax.ShapeDtypeStruct((seq, te, tb, 8, 128), jnp.float32),
    )


def kernel(x, embed_weight):
    b, s = x.shape
    n_tags, d = embed_weight.shape
    g = 128 // d
    sg = s // g
    tb = b // 128
    flat = x.reshape(b * s).astype(jnp.int32)
    tab = embed_weight.astype(jnp.float32)
    p1 = _make_gather(b * s, n_tags, d, s)(flat, tab)
    p3 = p1.reshape(tb, sg * 128, 128)
    o5 = _make_transpose(b, s, d)(p3)
    return o5.transpose(2, 4, 0, 1, 3).reshape(b, s, d)
